# baseline (speedup 1.0000x reference)
import functools

import jax
import jax.numpy as jnp
from jax import lax
from jax.experimental import pallas as pl
from jax.experimental.pallas import tpu as pltpu

E = 8
K = 2
DIM = 384
PD = 64
HID = 1536
NH = 4
HD = PD // NH

NTOK = 784
NPAIR = NTOK * K
RB = 224
NB = NPAIR // RB

_HI = lax.Precision.HIGHEST


def _iota(shape, axis):
    return lax.broadcasted_iota(jnp.int32, shape, axis)


def _mega_kernel(nbatch, hw,
                 x_ref, pr_ref, wcp_ref, bcp_ref, wip_ref, bip_ref,
                 wqkv_ref, bqkv_ref, wo_ref, bo_ref, lng_ref, lnb_ref,
                 w1e_ref, b1e_ref, w2e_ref, b2e_ref,
                 out_ref, total_ref,
                 qt_ref, cb_ref, xs_ref, ys_ref, xfs_ref, offs_ref,
                 w1b_ref, w2b_ref):
    s = pl.program_id(0)
    L = E + 1
    BL = nbatch * L

    @pl.when(s == 0)
    def _router():
        eyeh = jnp.where(_iota((hw, hw), 0) == _iota((hw, hw), 1), 1.0, 0.0)
        xf = jnp.concatenate(
            [lax.dot_general(eyeh, x_ref[b * DIM:(b + 1) * DIM, :],
                             (((1,), (1,)), ((), ())))
             for b in range(nbatch)], axis=0)
        xfs_ref[...] = xf

        rowb = _iota((nbatch, NTOK), 1) // hw
        bsel = jnp.where(rowb == _iota((nbatch, NTOK), 0), 1.0 / hw, 0.0)
        xmean = jnp.dot(bsel, xf, precision=_HI)
        gc = jnp.dot(xmean, wcp_ref[...].T) + bcp_ref[...]

        r_i = _iota((BL, nbatch), 0)
        g1 = jnp.where(r_i == _iota((BL, nbatch), 1) * L, 1.0, 0.0)
        r_i2 = _iota((BL, E), 0) % L
        g2 = jnp.where(r_i2 == _iota((BL, E), 1) + 1, 1.0, 0.0)
        seq = (jnp.dot(g1, gc, precision=_HI)
               + jnp.dot(g2, pr_ref[...], precision=_HI))

        qkv = jnp.dot(seq, wqkv_ref[...].T) + bqkv_ref[...]
        q = qkv[:, 0:PD]
        k = qkv[:, PD:2 * PD]
        v = qkv[:, 2 * PD:3 * PD]
        same_b = (_iota((BL, BL), 0) // L) == (_iota((BL, BL), 1) // L)
        ctxs = []
        for h in range(NH):
            qh = q[:, h * HD:(h + 1) * HD]
            kh = k[:, h * HD:(h + 1) * HD]
            vh = v[:, h * HD:(h + 1) * HD]
            sc = jnp.dot(qh, kh.T) * (1.0 / (HD ** 0.5))
            sc = jnp.where(same_b, sc, -1e30)
            m = jnp.max(sc, axis=-1, keepdims=True)
            p = jnp.exp(sc - m)
            p = p / jnp.sum(p, axis=-1, keepdims=True)
            ctxs.append(jnp.dot(p, vh))
        ctx = jnp.concatenate(ctxs, axis=1)
        y = jnp.dot(ctx, wo_ref[...].T) + bo_ref[...] + seq
        mu = jnp.mean(y, axis=-1, keepdims=True)
        var = jnp.mean((y - mu) * (y - mu), axis=-1, keepdims=True)
        y = (y - mu) * lax.rsqrt(var + 1e-5) * lng_ref[...] + lnb_ref[...]

        r_u = _iota((nbatch * E, BL), 1)
        usel = jnp.where(
            r_u == (_iota((nbatch * E, BL), 0) // E) * L
            + (_iota((nbatch * E, BL), 0) % E) + 1, 1.0, 0.0)
        upd = jnp.dot(usel, y, precision=_HI)
        xproj = jnp.dot(xf, wip_ref[...].T) + bip_ref[...]
        logits_all = jnp.dot(xproj, upd.T) * (1.0 / (PD ** 0.5))
        tokb = _iota((NTOK, 1), 0) // hw
        logits = jnp.zeros((NTOK, E), jnp.float32)
        for b in range(nbatch):
            logits = logits + jnp.where(
                tokb == b, logits_all[:, b * E:(b + 1) * E], 0.0)

        m = jnp.max(logits, axis=-1, keepdims=True)
        ex = jnp.exp(logits - m)
        probs = ex / jnp.sum(ex, axis=-1, keepdims=True)
        eio = _iota((NTOK, E), 1)
        m1 = jnp.max(probs, axis=-1, keepdims=True)
        i1 = jnp.min(jnp.where(probs == m1, eio, E), axis=-1, keepdims=True)
        pm = jnp.where(eio == i1, -1.0, probs)
        m2 = jnp.max(pm, axis=-1, keepdims=True)
        i2 = jnp.min(jnp.where(pm == m2, eio, E), axis=-1, keepdims=True)
        ssum = m1 + m2
        w1 = m1 / ssum
        w2 = m2 / ssum

        oh1 = jnp.where(eio == i1, 1.0, 0.0)
        oh2 = jnp.where(eio == i2, 1.0, 0.0)
        cnt1 = jnp.sum(oh1, axis=0, keepdims=True)
        cnt2 = jnp.sum(oh2, axis=0, keepdims=True)
        cnt = cnt1 + cnt2
        tri_e = jnp.where(_iota((E, E), 0) < _iota((E, E), 1), 1.0, 0.0)
        offs = jnp.dot(cnt, tri_e, precision=_HI)
        tri_t = jnp.where(_iota((NTOK, NTOK), 0) > _iota((NTOK, NTOK), 1),
                          1.0, 0.0).astype(jnp.bfloat16)
        oh12 = jnp.concatenate([oh1, oh2], axis=1).astype(jnp.bfloat16)
        ranks = lax.dot_general(tri_t, oh12, (((1,), (0,)), ((), ())),
                                preferred_element_type=jnp.float32)
        rank1 = jnp.sum(ranks[:, :E] * oh1, axis=-1, keepdims=True)
        rank2 = jnp.sum(ranks[:, E:] * oh2, axis=-1, keepdims=True)
        pos1 = jnp.sum(oh1 * offs, axis=-1, keepdims=True) + rank1
        pos2 = jnp.sum(oh2 * (offs + cnt1), axis=-1, keepdims=True) + rank2
        pos1 = pos1.astype(jnp.int32)
        pos2 = pos2.astype(jnp.int32)
        offs_ref[...] = jnp.concatenate(
            [offs, jnp.full((1, 1), NPAIR, jnp.float32),
             jnp.zeros((1, 16 - E - 1), jnp.float32)], axis=1)

        s_io = _iota((NTOK, NPAIR), 1)
        eq1 = s_io == pos1
        eq2 = s_io == pos2
        qt_ref[...] = jnp.where(eq1 | eq2, 1.0, 0.0).astype(jnp.bfloat16)
        cb_ref[...] = (jnp.where(eq1, w1, 0.0)
                       + jnp.where(eq2, w2, 0.0)).astype(jnp.bfloat16)
        xs_ref[...] = lax.dot_general(
            qt_ref[...], xf.astype(jnp.bfloat16), (((0,), (0,)), ((), ())),
            preferred_element_type=jnp.float32).astype(jnp.bfloat16)
        ys_ref[...] = jnp.zeros((NPAIR, DIM), jnp.bfloat16)

        mean_prob = jnp.mean(probs, axis=0, keepdims=True)
        mean_load = jnp.mean(oh1 + oh2, axis=0, keepdims=True)
        aux = E * jnp.sum(mean_prob * mean_load, axis=-1, keepdims=True)
        pr = pr_ref[...]
        nrm = jnp.sqrt(jnp.sum(pr * pr, axis=-1, keepdims=True))
        pn = pr / jnp.maximum(nrm, 1e-12)
        corr = jnp.dot(pn, pn.T)
        eye = jnp.where(_iota((E, E), 0) == _iota((E, E), 1), 1.0, 0.0)
        d = corr - eye
        sq = jnp.sum(jnp.sum(d * d, axis=-1, keepdims=True),
                     axis=0, keepdims=True)
        total_ref[...] = aux + 0.5 * jnp.sqrt(sq)

    kk = jnp.maximum(s - 1, 0)
    e = kk // NB
    b = kk % NB

    prev_e = jnp.maximum(s - 2, 0) // NB

    @pl.when((s == 1) | ((s >= 2) & (e != prev_e)))
    def _cast_w():
        w1b_ref[...] = w1e_ref[0].astype(jnp.bfloat16)
        w2b_ref[...] = w2e_ref[0].astype(jnp.bfloat16)

    lane = _iota((1, 16), 1)
    offs_row = offs_ref[...]
    start = jnp.sum(jnp.where(lane == e, offs_row, 0.0))
    end = jnp.sum(jnp.where(lane == e + 1, offs_row, 0.0))
    blk_lo = b * RB
    active = ((s >= 1) & (start < blk_lo + RB)
              & (end > blk_lo) & (end > start))

    @pl.when(active)
    def _ffn():
        rows = xs_ref[pl.ds(blk_lo, RB), :]
        h = lax.dot_general(rows, w1b_ref[...], (((1,), (1,)), ((), ())),
                            preferred_element_type=jnp.float32)
        h = h + b1e_ref[0]
        t = jnp.tanh(0.7978845608028654 * (h + 0.044715 * h * h * h))
        h = (0.5 * h * (1.0 + t)).astype(jnp.bfloat16)
        o = lax.dot_general(h, w2b_ref[...], (((1,), (1,)), ((), ())),
                            preferred_element_type=jnp.float32)
        o = o + b2e_ref[0]
        r_io = _iota((RB, 1), 0) + blk_lo
        msk = (r_io >= start) & (r_io < end)
        ys_ref[pl.ds(blk_lo, RB), :] += jnp.where(
            msk, o, 0.0).astype(jnp.bfloat16)

    @pl.when(s == E * NB)
    def _combine():
        fin = xfs_ref[...] + lax.dot_general(
            cb_ref[...], ys_ref[...], (((1,), (0,)), ((), ())),
            preferred_element_type=jnp.float32)
        eyed = jnp.where(_iota((DIM, DIM), 0) == _iota((DIM, DIM), 1),
                         1.0, 0.0)
        outs = [lax.dot_general(eyed, fin[b2 * hw:(b2 + 1) * hw, :],
                                (((1,), (1,)), ((), ())))
                for b2 in range(nbatch)]
        out_ref[...] = jnp.concatenate(outs, axis=0)


def kernel(x, prototypes, W_cp, b_cp, W_ip, b_ip, Wqkv, bqkv, Wo, bo,
           ln_g, ln_b, W1, b1, W2, b2):
    B, C, H, W = x.shape
    hw = H * W
    x_r = x.reshape(B * C, hw)

    def we_map(s):
        return (jnp.maximum(s - 1, 0) // NB, 0, 0)

    mega = pl.pallas_call(
        functools.partial(_mega_kernel, B, hw),
        grid=(1 + E * NB,),
        in_specs=[
            pl.BlockSpec((B * C, hw), lambda s: (0, 0)),
            pl.BlockSpec((E, PD), lambda s: (0, 0)),
            pl.BlockSpec((PD, DIM), lambda s: (0, 0)),
            pl.BlockSpec((1, PD), lambda s: (0, 0)),
            pl.BlockSpec((PD, DIM), lambda s: (0, 0)),
            pl.BlockSpec((1, PD), lambda s: (0, 0)),
            pl.BlockSpec((3 * PD, PD), lambda s: (0, 0)),
            pl.BlockSpec((1, 3 * PD), lambda s: (0, 0)),
            pl.BlockSpec((PD, PD), lambda s: (0, 0)),
            pl.BlockSpec((1, PD), lambda s: (0, 0)),
            pl.BlockSpec((1, PD), lambda s: (0, 0)),
            pl.BlockSpec((1, PD), lambda s: (0, 0)),
            pl.BlockSpec((1, HID, DIM), we_map),
            pl.BlockSpec((1, 1, HID), we_map),
            pl.BlockSpec((1, DIM, HID), we_map),
            pl.BlockSpec((1, 1, DIM), we_map),
        ],
        out_specs=[
            pl.BlockSpec((B * C, hw), lambda s: (0, 0)),
            pl.BlockSpec((1, 1), lambda s: (0, 0)),
        ],
        out_shape=[
            jax.ShapeDtypeStruct((B * C, hw), jnp.float32),
            jax.ShapeDtypeStruct((1, 1), jnp.float32),
        ],
        scratch_shapes=[
            pltpu.VMEM((NTOK, NPAIR), jnp.bfloat16),
            pltpu.VMEM((NTOK, NPAIR), jnp.bfloat16),
            pltpu.VMEM((NPAIR, DIM), jnp.bfloat16),
            pltpu.VMEM((NPAIR, DIM), jnp.bfloat16),
            pltpu.VMEM((NTOK, DIM), jnp.float32),
            pltpu.VMEM((1, 16), jnp.float32),
            pltpu.VMEM((HID, DIM), jnp.bfloat16),
            pltpu.VMEM((DIM, HID), jnp.bfloat16),
        ],
        compiler_params=pltpu.CompilerParams(
            dimension_semantics=("arbitrary",)),
    )
    out, total = mega(
        x_r, prototypes, W_cp, b_cp.reshape(1, PD), W_ip,
        b_ip.reshape(1, PD), Wqkv, bqkv.reshape(1, 3 * PD), Wo,
        bo.reshape(1, PD), ln_g.reshape(1, PD), ln_b.reshape(1, PD),
        W1, b1.reshape(E, 1, HID), W2, b2.reshape(E, 1, DIM))

    y = out.reshape(B, C, H, W)
    return y, total[0, 0]


# speedup vs baseline: 1.1405x; 1.1405x over previous
import functools

import jax
import jax.numpy as jnp
from jax import lax
from jax.experimental import pallas as pl
from jax.experimental.pallas import tpu as pltpu

E = 8
K = 2
DIM = 384
PD = 64
HID = 1536
NH = 4
HD = PD // NH

NTOK = 784
NPAIR = NTOK * K
RB = 112
NB = NPAIR // RB
NITEM = NB + E - 1

_HI = lax.Precision.HIGHEST


def _iota(shape, axis):
    return lax.broadcasted_iota(jnp.int32, shape, axis)


def _router_kernel(nbatch, hw,
                   x_ref, pr_ref, wcp_ref, bcp_ref, wip_ref, bip_ref,
                   wqkv_ref, bqkv_ref, wo_ref, bo_ref, lng_ref, lnb_ref,
                   pos1_ref, pos2_ref, w1_ref, w2_ref, offs_ref, meta_ref,
                   xf_ref, total_ref):
    eyeh = jnp.where(_iota((hw, hw), 0) == _iota((hw, hw), 1), 1.0, 0.0)
    xf = jnp.concatenate(
        [lax.dot_general(eyeh, x_ref[b * DIM:(b + 1) * DIM, :],
                         (((1,), (1,)), ((), ())))
         for b in range(nbatch)], axis=0)
    xf_ref[...] = xf
    L = E + 1
    BL = nbatch * L

    rowb = _iota((nbatch, NTOK), 1) // hw
    bsel = jnp.where(rowb == _iota((nbatch, NTOK), 0), 1.0 / hw, 0.0)
    xmean = jnp.dot(bsel, xf, precision=_HI)
    gc = jnp.dot(xmean, wcp_ref[...].T) + bcp_ref[...]

    r_i = _iota((BL, nbatch), 0)
    g1 = jnp.where(r_i == _iota((BL, nbatch), 1) * L, 1.0, 0.0)
    r_i2 = _iota((BL, E), 0) % L
    g2 = jnp.where(r_i2 == _iota((BL, E), 1) + 1, 1.0, 0.0)
    seq = jnp.dot(g1, gc, precision=_HI) + jnp.dot(g2, pr_ref[...], precision=_HI)

    qkv = jnp.dot(seq, wqkv_ref[...].T) + bqkv_ref[...]
    q = qkv[:, 0:PD]
    k = qkv[:, PD:2 * PD]
    v = qkv[:, 2 * PD:3 * PD]
    same_b = (_iota((BL, BL), 0) // L) == (_iota((BL, BL), 1) // L)
    ctxs = []
    for h in range(NH):
        qh = q[:, h * HD:(h + 1) * HD]
        kh = k[:, h * HD:(h + 1) * HD]
        vh = v[:, h * HD:(h + 1) * HD]
        sc = jnp.dot(qh, kh.T) * (1.0 / (HD ** 0.5))
        sc = jnp.where(same_b, sc, -1e30)
        m = jnp.max(sc, axis=-1, keepdims=True)
        p = jnp.exp(sc - m)
        p = p / jnp.sum(p, axis=-1, keepdims=True)
        ctxs.append(jnp.dot(p, vh))
    ctx = jnp.concatenate(ctxs, axis=1)
    y = jnp.dot(ctx, wo_ref[...].T) + bo_ref[...] + seq
    mu = jnp.mean(y, axis=-1, keepdims=True)
    var = jnp.mean((y - mu) * (y - mu), axis=-1, keepdims=True)
    y = (y - mu) * lax.rsqrt(var + 1e-5) * lng_ref[...] + lnb_ref[...]

    r_u = _iota((nbatch * E, BL), 1)
    usel = jnp.where(
        r_u == (_iota((nbatch * E, BL), 0) // E) * L + (_iota((nbatch * E, BL), 0) % E) + 1,
        1.0, 0.0)
    upd = jnp.dot(usel, y, precision=_HI)
    xproj = jnp.dot(xf, wip_ref[...].T) + bip_ref[...]
    logits_all = jnp.dot(xproj, upd.T) * (1.0 / (PD ** 0.5))
    tokb = _iota((NTOK, 1), 0) // hw
    logits = jnp.zeros((NTOK, E), jnp.float32)
    for b in range(nbatch):
        logits = logits + jnp.where(tokb == b, logits_all[:, b * E:(b + 1) * E], 0.0)

    m = jnp.max(logits, axis=-1, keepdims=True)
    ex = jnp.exp(logits - m)
    probs = ex / jnp.sum(ex, axis=-1, keepdims=True)
    eio = _iota((NTOK, E), 1)
    m1 = jnp.max(probs, axis=-1, keepdims=True)
    i1 = jnp.min(jnp.where(probs == m1, eio, E), axis=-1, keepdims=True)
    pm = jnp.where(eio == i1, -1.0, probs)
    m2 = jnp.max(pm, axis=-1, keepdims=True)
    i2 = jnp.min(jnp.where(pm == m2, eio, E), axis=-1, keepdims=True)
    ssum = m1 + m2
    w1_ref[...] = m1 / ssum
    w2_ref[...] = m2 / ssum

    oh1 = jnp.where(eio == i1, 1.0, 0.0)
    oh2 = jnp.where(eio == i2, 1.0, 0.0)
    cnt1 = jnp.sum(oh1, axis=0, keepdims=True)
    cnt2 = jnp.sum(oh2, axis=0, keepdims=True)
    cnt = cnt1 + cnt2
    tri_e = jnp.where(_iota((E, E), 0) < _iota((E, E), 1), 1.0, 0.0)
    offs = jnp.dot(cnt, tri_e, precision=_HI)
    tri_t = jnp.where(_iota((NTOK, NTOK), 0) > _iota((NTOK, NTOK), 1),
                      1.0, 0.0).astype(jnp.bfloat16)
    oh12 = jnp.concatenate([oh1, oh2], axis=1).astype(jnp.bfloat16)
    ranks = lax.dot_general(tri_t, oh12, (((1,), (0,)), ((), ())),
                            preferred_element_type=jnp.float32)
    rank1 = jnp.sum(ranks[:, :E] * oh1, axis=-1, keepdims=True)
    rank2 = jnp.sum(ranks[:, E:] * oh2, axis=-1, keepdims=True)
    pos1 = jnp.sum(oh1 * offs, axis=-1, keepdims=True) + rank1
    pos2 = jnp.sum(oh2 * (offs + cnt1), axis=-1, keepdims=True) + rank2
    pos1_ref[...] = pos1.astype(jnp.int32)
    pos2_ref[...] = pos2.astype(jnp.int32)
    offs16 = jnp.concatenate(
        [offs, jnp.full((1, 1), NPAIR, jnp.float32),
         jnp.zeros((1, 16 - E - 1), jnp.float32)], axis=1)
    offs_ref[...] = offs16.astype(jnp.int32)

    ends = jnp.concatenate([offs[:, 1:], jnp.full((1, 1), float(NPAIR))], axis=1)
    nc = E * NB
    ce = _iota((nc, 1), 0) // NB
    cb = _iota((nc, 1), 0) % NB
    ohe = jnp.where(_iota((nc, E), 1) == ce, 1.0, 0.0)
    st = jnp.sum(ohe * offs, axis=-1, keepdims=True)
    en = jnp.sum(ohe * ends, axis=-1, keepdims=True)
    cbf = cb.astype(jnp.float32)
    act = (st < (cbf + 1.0) * RB) & (en > cbf * RB) & (en > st)
    actf = jnp.where(act, 1.0, 0.0)
    tri_c = jnp.where(_iota((nc, nc), 0) > _iota((nc, nc), 1), 1.0, 0.0)
    ordv = jnp.dot(tri_c, actf, precision=_HI)
    n_act = jnp.sum(actf, axis=0, keepdims=True)
    s_io2 = _iota((nc, NITEM), 1).astype(jnp.float32)
    target = jnp.minimum(s_io2, n_act - 1.0)
    sel = jnp.where((ordv == target) & act, 1.0, 0.0)
    item_e = lax.dot_general(ce.astype(jnp.float32), sel,
                             (((0,), (0,)), ((), ())), precision=_HI)
    item_b = lax.dot_general(cbf, sel, (((0,), (0,)), ((), ())), precision=_HI)
    item_v = jnp.where(_iota((1, NITEM), 1).astype(jnp.float32) < n_act, 1.0, 0.0)
    meta = jnp.concatenate(
        [item_e, item_b, item_v, jnp.zeros((1, NITEM), jnp.float32)], axis=0)
    meta_ref[...] = meta.astype(jnp.int32)

    mean_prob = jnp.mean(probs, axis=0, keepdims=True)
    mean_load = jnp.mean(oh1 + oh2, axis=0, keepdims=True)
    aux = E * jnp.sum(mean_prob * mean_load, axis=-1, keepdims=True)
    pr = pr_ref[...]
    nrm = jnp.sqrt(jnp.sum(pr * pr, axis=-1, keepdims=True))
    pn = pr / jnp.maximum(nrm, 1e-12)
    corr = jnp.dot(pn, pn.T)
    eye = jnp.where(_iota((E, E), 0) == _iota((E, E), 1), 1.0, 0.0)
    d = corr - eye
    sq = jnp.sum(jnp.sum(d * d, axis=-1, keepdims=True), axis=0, keepdims=True)
    total_ref[...] = aux + 0.5 * jnp.sqrt(sq)


def _moe_kernel(nbatch, hw,
                meta_ref, offs_ref, xf_ref, pos1_ref, pos2_ref,
                w1_ref, w2_ref, w1e_ref, b1e_ref, w2e_ref, b2e_ref,
                out_ref, qt_ref, cb_ref, xs_ref, ys_ref, w1b_ref, w2b_ref):
    s = pl.program_id(0)
    e = meta_ref[0, s]
    b = meta_ref[1, s]
    valid = meta_ref[2, s]

    @pl.when(s == 0)
    def _gather():
        s_io = _iota((NTOK, NPAIR), 1)
        eq1 = s_io == pos1_ref[...]
        eq2 = s_io == pos2_ref[...]
        qt_ref[...] = jnp.where(eq1 | eq2, 1.0, 0.0).astype(jnp.bfloat16)
        cb_ref[...] = (jnp.where(eq1, w1_ref[...], 0.0)
                       + jnp.where(eq2, w2_ref[...], 0.0)).astype(jnp.bfloat16)
        xs_ref[...] = lax.dot_general(
            qt_ref[...], xf_ref[...].astype(jnp.bfloat16),
            (((0,), (0,)), ((), ())),
            preferred_element_type=jnp.float32).astype(jnp.bfloat16)
        ys_ref[...] = jnp.zeros((NPAIR, DIM), jnp.bfloat16)

    @pl.when((s == 0) | (e != meta_ref[0, jnp.maximum(s - 1, 0)]))
    def _cast_w():
        w1b_ref[...] = w1e_ref[0].astype(jnp.bfloat16)
        w2b_ref[...] = w2e_ref[0].astype(jnp.bfloat16)

    start = offs_ref[0, e]
    end = offs_ref[0, e + 1]
    blk_lo = b * RB

    @pl.when(valid == 1)
    def _ffn():
        rows = xs_ref[pl.ds(blk_lo, RB), :]
        h = lax.dot_general(rows, w1b_ref[...], (((1,), (1,)), ((), ())),
                            preferred_element_type=jnp.float32)
        h = h + b1e_ref[0]
        t = jnp.tanh(0.7978845608028654 * (h + 0.044715 * h * h * h))
        h = (0.5 * h * (1.0 + t)).astype(jnp.bfloat16)
        o = lax.dot_general(h, w2b_ref[...], (((1,), (1,)), ((), ())),
                            preferred_element_type=jnp.float32)
        o = o + b2e_ref[0]
        r_io = _iota((RB, 1), 0) + blk_lo
        msk = (r_io >= start) & (r_io < end)
        ys_ref[pl.ds(blk_lo, RB), :] += jnp.where(
            msk, o, 0.0).astype(jnp.bfloat16)

    @pl.when(s == NITEM - 1)
    def _combine():
        fin = xf_ref[...] + lax.dot_general(
            cb_ref[...], ys_ref[...], (((1,), (0,)), ((), ())),
            preferred_element_type=jnp.float32)
        eyed = jnp.where(_iota((DIM, DIM), 0) == _iota((DIM, DIM), 1),
                         1.0, 0.0)
        outs = [lax.dot_general(eyed, fin[b2 * hw:(b2 + 1) * hw, :],
                                (((1,), (1,)), ((), ())))
                for b2 in range(nbatch)]
        out_ref[...] = jnp.concatenate(outs, axis=0)


def kernel(x, prototypes, W_cp, b_cp, W_ip, b_ip, Wqkv, bqkv, Wo, bo,
           ln_g, ln_b, W1, b1, W2, b2):
    B, C, H, W = x.shape
    hw = H * W
    x_r = x.reshape(B * C, hw)

    router = pl.pallas_call(
        functools.partial(_router_kernel, B, hw),
        out_shape=[
            jax.ShapeDtypeStruct((NTOK, 1), jnp.int32),
            jax.ShapeDtypeStruct((NTOK, 1), jnp.int32),
            jax.ShapeDtypeStruct((NTOK, 1), jnp.float32),
            jax.ShapeDtypeStruct((NTOK, 1), jnp.float32),
            jax.ShapeDtypeStruct((1, 16), jnp.int32),
            jax.ShapeDtypeStruct((4, NITEM), jnp.int32),
            jax.ShapeDtypeStruct((NTOK, DIM), jnp.float32),
            jax.ShapeDtypeStruct((1, 1), jnp.float32),
        ],
    )
    pos1, pos2, w1, w2, offs, meta, xf, total = router(
        x_r, prototypes, W_cp, b_cp.reshape(1, PD), W_ip,
        b_ip.reshape(1, PD), Wqkv, bqkv.reshape(1, 3 * PD), Wo,
        bo.reshape(1, PD), ln_g.reshape(1, PD), ln_b.reshape(1, PD))

    moe = pl.pallas_call(
        functools.partial(_moe_kernel, B, hw),
        grid_spec=pltpu.PrefetchScalarGridSpec(
            num_scalar_prefetch=2,
            grid=(NITEM,),
            in_specs=[
                pl.BlockSpec((NTOK, DIM), lambda s, m, o: (0, 0)),
                pl.BlockSpec((NTOK, 1), lambda s, m, o: (0, 0)),
                pl.BlockSpec((NTOK, 1), lambda s, m, o: (0, 0)),
                pl.BlockSpec((NTOK, 1), lambda s, m, o: (0, 0)),
                pl.BlockSpec((NTOK, 1), lambda s, m, o: (0, 0)),
                pl.BlockSpec((1, HID, DIM), lambda s, m, o: (m[0, s], 0, 0)),
                pl.BlockSpec((1, 1, HID), lambda s, m, o: (m[0, s], 0, 0)),
                pl.BlockSpec((1, DIM, HID), lambda s, m, o: (m[0, s], 0, 0)),
                pl.BlockSpec((1, 1, DIM), lambda s, m, o: (m[0, s], 0, 0)),
            ],
            out_specs=pl.BlockSpec((B * C, hw), lambda s, m, o: (0, 0)),
            scratch_shapes=[
                pltpu.VMEM((NTOK, NPAIR), jnp.bfloat16),
                pltpu.VMEM((NTOK, NPAIR), jnp.bfloat16),
                pltpu.VMEM((NPAIR, DIM), jnp.bfloat16),
                pltpu.VMEM((NPAIR, DIM), jnp.bfloat16),
                pltpu.VMEM((HID, DIM), jnp.bfloat16),
                pltpu.VMEM((DIM, HID), jnp.bfloat16),
            ],
        ),
        out_shape=jax.ShapeDtypeStruct((B * C, hw), jnp.float32),
        compiler_params=pltpu.CompilerParams(
            dimension_semantics=("arbitrary",)),
    )
    out = moe(meta, offs, xf, pos1, pos2, w1, w2, W1,
              b1.reshape(E, 1, HID), W2, b2.reshape(E, 1, DIM))

    y = out.reshape(B, C, H, W)
    return y, total[0, 0]
